# baseline (speedup 1.0000x reference)
import functools

import jax
import jax.numpy as jnp
from jax import lax
from jax.experimental import pallas as pl
from jax.experimental.pallas import tpu as pltpu
from jax.experimental.pallas import tpu_sc as plsc

N = 10000
E = 320000
HID = 128
D1 = 16
DH = 64

NC, NS = 2, 16
NPAD = 10240
EPAD = 327680
EPT = EPAD // (NC * NS)
CH = 128
NCHUNK = EPT // CH
RPT = NPAD // NS

_mesh = functools.partial(
    plsc.VectorSubcoreMesh, core_axis_name="c", subcore_axis_name="s",
    num_cores=NC, num_subcores=NS)


RB2, GD2 = 3, 2


def _sc_agg2():
  scratch = [
      pltpu.VMEM((NCHUNK, CH), jnp.int32),
      pltpu.VMEM((NCHUNK, CH), jnp.int32),
      pltpu.VMEM((RB2, CH, DH), jnp.float32),
      pltpu.VMEM_SHARED((NPAD, DH), jnp.float32),
      pltpu.VMEM_SHARED((NPAD, DH), jnp.float32),
      pltpu.SemaphoreType.DMA((RB2,)),
      pltpu.SemaphoreType.DMA((RB2,)),
  ]

  def body(ga, gb, src3, dst3, outa, outb,
           src_v, dst_v, rows_v, accum, table_sp, ssem, gsem):
    c = lax.axis_index("c")
    s = lax.axis_index("s")
    wid = c * NS + s
    r0 = pl.multiple_of(s * RPT, RPT)

    def zero_fill(r, carry):
      for j in range(DH // 16):
        rows_v[0, r, pl.ds(j * 16, 16)] = jnp.zeros((16,), jnp.float32)
      return carry

    def scat_wait(b):
      pltpu.make_async_copy(
          rows_v.at[0], accum.at[dst_v.at[0]], ssem.at[b]).wait()

    def gath_wait(b):
      pltpu.make_async_copy(
          table_sp.at[src_v.at[0]], rows_v.at[0], gsem.at[b]).wait()

    def gath_start(i):
      b = lax.rem(i, RB2)
      pltpu.async_copy(table_sp.at[src_v.at[i]], rows_v.at[b], gsem.at[b])

    pltpu.sync_copy(src3.at[wid], src_v)
    pltpu.sync_copy(dst3.at[wid], dst_v)

    for table, out in ((ga, outa), (gb, outb)):
      lax.fori_loop(0, CH, zero_fill, 0)
      for k in range(RPT // CH):
        pltpu.sync_copy(rows_v.at[0], accum.at[pl.ds(r0 + k * CH, CH)])
      pltpu.sync_copy(table.at[pl.ds(r0, RPT)], table_sp.at[pl.ds(r0, RPT)])
      plsc.subcore_barrier()

      for j in range(GD2):
        gath_start(j)

      def step(i, carry):
        b = lax.rem(i, RB2)
        gath_wait(b)
        pltpu.async_copy(
            rows_v.at[b], accum.at[dst_v.at[i]], ssem.at[b], add=True)
        j = i + GD2

        @pl.when(j < NCHUNK)
        def _():
          @pl.when(j >= RB2)
          def _():
            scat_wait(lax.rem(j, RB2))
          gath_start(j)
        return carry

      lax.fori_loop(0, NCHUNK, step, 0)
      for b in range(RB2):
        scat_wait(b)
      plsc.subcore_barrier()
      pltpu.sync_copy(accum.at[pl.ds(r0, RPT)], out.at[c, pl.ds(r0, RPT)])

  return pl.kernel(
      body,
      out_type=(jax.ShapeDtypeStruct((NC, NPAD, DH), jnp.float32),
                jax.ShapeDtypeStruct((NC, NPAD, DH), jnp.float32)),
      mesh=_mesh(),
      scratch_types=scratch,
      compiler_params=pltpu.CompilerParams(use_tc_tiling_on_sc=False),
  )


NCHUNK_ALL = EPAD // NS // CH


def _sc_deg_agg1():
  rb, gd = 8, 5
  scratch = [
      pltpu.VMEM((NCHUNK_ALL, CH), jnp.int32),
      pltpu.VMEM((NCHUNK, CH), jnp.int32),
      pltpu.VMEM((NCHUNK, CH), jnp.int32),
      pltpu.VMEM((rb, CH, D1), jnp.float32),
      pltpu.VMEM((RPT, D1), jnp.float32),
      pltpu.VMEM((RPT, D1), jnp.float32),
      pltpu.VMEM_SHARED((NPAD, D1), jnp.float32),
      pltpu.VMEM_SHARED((NPAD, D1), jnp.float32),
      pltpu.VMEM_SHARED((NPAD, D1), jnp.float32),
      pltpu.SemaphoreType.DMA((rb,)),
      pltpu.SemaphoreType.DMA((rb,)),
  ]

  def body(x16, src3, dst3, g1_out, s1_out, dsta_v, src_v, dst_v,
           rows_v, work_v, deg_v, degacc, table, s1acc, ssem, gsem):
    c = lax.axis_index("c")
    s = lax.axis_index("s")
    wid = c * NS + s
    r0 = pl.multiple_of(s * RPT, RPT)

    def make_fill(val):
      def fill_row(r, carry):
        rows_v[0, r, :] = jnp.full((16,), val, jnp.float32)
        return carry
      return fill_row

    lax.fori_loop(0, CH, make_fill(0.0), 0)
    for k in range(RPT // CH):
      pltpu.sync_copy(rows_v.at[0], degacc.at[pl.ds(r0 + k * CH, CH)])
      pltpu.sync_copy(rows_v.at[0], s1acc.at[pl.ds(r0 + k * CH, CH)])
    lax.fori_loop(0, CH, make_fill(1.0), 0)
    pltpu.sync_copy(dst3.at[2 * s], dsta_v.at[pl.ds(0, NCHUNK)])
    pltpu.sync_copy(dst3.at[2 * s + 1], dsta_v.at[pl.ds(NCHUNK, NCHUNK)])
    pltpu.sync_copy(src3.at[wid], src_v)
    pltpu.sync_copy(dst3.at[wid], dst_v)
    pltpu.sync_copy(x16.at[pl.ds(r0, RPT)], work_v)
    plsc.subcore_barrier()

    def scat_wait(b):
      pltpu.make_async_copy(
          rows_v.at[0], s1acc.at[pl.ds(0, CH)], ssem.at[b]).wait()

    def deg_step(i, carry):
      b = lax.rem(i, rb)

      @pl.when(i >= rb)
      def _():
        scat_wait(b)
      pltpu.async_copy(
          rows_v.at[0], degacc.at[dsta_v.at[i]], ssem.at[b], add=True)
      return carry

    lax.fori_loop(0, NCHUNK_ALL, deg_step, 0)
    for b in range(rb):
      scat_wait(b)
    plsc.subcore_barrier()

    pltpu.sync_copy(degacc.at[pl.ds(r0, RPT)], deg_v)
    lane = lax.iota(jnp.int32, 16)
    magic = jnp.full((16,), 0x5F3759DF, jnp.int32)

    def norm_row(r, carry):
      d1 = deg_v[r, :] + 1.0
      y = plsc.bitcast(
          magic - lax.shift_right_logical(plsc.bitcast(d1, jnp.int32), 1),
          jnp.float32)
      for _ in range(3):
        y = y * (1.5 - 0.5 * d1 * y * y)
      y = jnp.where(r0 + r < N, y, 0.0)
      work_v[r, :] = jnp.where(lane == 4, y, work_v[r, :] * y)
      return carry

    lax.fori_loop(0, RPT, norm_row, 0)
    pltpu.sync_copy(work_v, table.at[pl.ds(r0, RPT)])

    @pl.when(c == 0)
    def _():
      pltpu.sync_copy(work_v, g1_out.at[pl.ds(r0, RPT)])
    plsc.subcore_barrier()

    def gath_start(i):
      b = lax.rem(i, rb)
      pltpu.async_copy(table.at[src_v.at[i]], rows_v.at[b], gsem.at[b])

    def gath_wait(b):
      pltpu.make_async_copy(
          table.at[pl.ds(0, CH)], rows_v.at[0], gsem.at[b]).wait()

    for j in range(gd):
      gath_start(j)

    def step(i, carry):
      b = lax.rem(i, rb)
      gath_wait(b)
      pltpu.async_copy(
          rows_v.at[b], s1acc.at[dst_v.at[i]], ssem.at[b], add=True)
      j = i + gd

      @pl.when(j < NCHUNK)
      def _():
        @pl.when(j >= rb)
        def _():
          scat_wait(lax.rem(j, rb))
        gath_start(j)
      return carry

    lax.fori_loop(0, NCHUNK, step, 0)
    for b in range(min(rb, NCHUNK)):
      scat_wait(b)
    plsc.subcore_barrier()
    pltpu.sync_copy(s1acc.at[pl.ds(r0, RPT)], s1_out.at[c, pl.ds(r0, RPT)])

  return pl.kernel(
      body,
      out_type=(jax.ShapeDtypeStruct((NPAD, D1), jnp.float32),
                jax.ShapeDtypeStruct((NC, NPAD, D1), jnp.float32)),
      mesh=_mesh(),
      scratch_types=scratch,
      compiler_params=pltpu.CompilerParams(
          use_tc_tiling_on_sc=False, needs_layout_passes=False),
  )


BR = 512
_GRID = NPAD // BR


def _layer1_body(g1_ref, s1_ref, x_ref, w_ref, b_ref, h1_ref, g2a_ref,
                 g2b_ref):
  dinv = g1_ref[:, 4:5]
  a1 = (s1_ref[0] + s1_ref[1]) * dinv + x_ref[...] * (dinv * dinv)
  h1 = jnp.maximum(
      jnp.dot(a1, w_ref[...], preferred_element_type=jnp.float32)
      + b_ref[...], 0.0)
  h1_ref[...] = h1
  g2 = h1 * dinv
  g2a_ref[...] = g2[:, :DH]
  g2b_ref[...] = g2[:, DH:]


def _tc_layer1(g1, s1_parts, x16, w1p, b1):
  return pl.pallas_call(
      _layer1_body,
      grid=(_GRID,),
      in_specs=[
          pl.BlockSpec((BR, D1), lambda i: (i, 0)),
          pl.BlockSpec((NC, BR, D1), lambda i: (0, i, 0)),
          pl.BlockSpec((BR, D1), lambda i: (i, 0)),
          pl.BlockSpec((D1, HID), lambda i: (0, 0)),
          pl.BlockSpec((1, HID), lambda i: (0, 0)),
      ],
      out_specs=[
          pl.BlockSpec((BR, HID), lambda i: (i, 0)),
          pl.BlockSpec((BR, DH), lambda i: (i, 0)),
          pl.BlockSpec((BR, DH), lambda i: (i, 0)),
      ],
      out_shape=[
          jax.ShapeDtypeStruct((NPAD, HID), jnp.float32),
          jax.ShapeDtypeStruct((NPAD, DH), jnp.float32),
          jax.ShapeDtypeStruct((NPAD, DH), jnp.float32),
      ],
  )(g1, s1_parts, x16, w1p, b1)


def _layer2_body(g1_ref, s2a_ref, s2b_ref, h1_ref, w_ref, b_ref, y_ref):
  dinv = g1_ref[:, 4:5]
  s2 = jnp.concatenate(
      [s2a_ref[0] + s2a_ref[1], s2b_ref[0] + s2b_ref[1]], axis=1)
  a2 = s2 * dinv + h1_ref[...] * (dinv * dinv)
  y_ref[...] = jnp.maximum(
      jnp.dot(a2, w_ref[...], preferred_element_type=jnp.float32)
      + b_ref[...], 0.0)


def _tc_layer2(g1, s2a_parts, s2b_parts, h1, w2, b2):
  return pl.pallas_call(
      _layer2_body,
      grid=(_GRID,),
      in_specs=[
          pl.BlockSpec((BR, D1), lambda i: (i, 0)),
          pl.BlockSpec((NC, BR, DH), lambda i: (0, i, 0)),
          pl.BlockSpec((NC, BR, DH), lambda i: (0, i, 0)),
          pl.BlockSpec((BR, HID), lambda i: (i, 0)),
          pl.BlockSpec((HID, HID), lambda i: (0, 0)),
          pl.BlockSpec((1, HID), lambda i: (0, 0)),
      ],
      out_specs=pl.BlockSpec((BR, HID), lambda i: (i, 0)),
      out_shape=jax.ShapeDtypeStruct((NPAD, HID), jnp.float32),
  )(g1, s2a_parts, s2b_parts, h1, w2, b2)


@jax.jit
def _run(x, edge_index, w1, b1, w2, b2):
  pad = jnp.full((EPAD - E,), N, dtype=jnp.int32)
  srcf = jnp.concatenate([edge_index[0].astype(jnp.int32), pad])
  dstf = jnp.concatenate([edge_index[1].astype(jnp.int32), pad])
  srcp = srcf.reshape(NC * NS, NCHUNK, CH)
  dstp = dstf.reshape(NC * NS, NCHUNK, CH)
  x16 = jnp.zeros((NPAD, D1), jnp.float32).at[:N, :4].set(x)
  w1p = jnp.zeros((D1, HID), jnp.float32).at[:4].set(w1)

  g1, s1_parts = _sc_deg_agg1()(x16, srcp, dstp)
  h1, g2a, g2b = _tc_layer1(g1, s1_parts, x16, w1p, b1.reshape(1, HID))
  s2a_parts, s2b_parts = _sc_agg2()(g2a, g2b, srcp, dstp)
  y = _tc_layer2(g1, s2a_parts, s2b_parts, h1, w2, b2.reshape(1, HID))
  return y[:N]


def kernel(x, edge_index, W1, b1, W2, b2):
  return _run(x, edge_index, W1, b1, W2, b2)


# speedup vs baseline: 1.0056x; 1.0056x over previous
import functools

import jax
import jax.numpy as jnp
from jax import lax
from jax.experimental import pallas as pl
from jax.experimental.pallas import tpu as pltpu
from jax.experimental.pallas import tpu_sc as plsc

N = 10000
E = 320000
HID = 128
D1 = 16
DH = 64

NC, NS = 2, 16
NPAD = 10240
EPAD = 327680
EPT = EPAD // (NC * NS)
CH = 128
NCHUNK = EPT // CH
RPT = NPAD // NS

_mesh = functools.partial(
    plsc.VectorSubcoreMesh, core_axis_name="c", subcore_axis_name="s",
    num_cores=NC, num_subcores=NS)


RB2, GD2 = 3, 2


def _sc_agg2():
  scratch = [
      pltpu.VMEM((NCHUNK, CH), jnp.int32),
      pltpu.VMEM((NCHUNK, CH), jnp.int32),
      pltpu.VMEM((RB2, CH, DH), jnp.float32),
      pltpu.VMEM_SHARED((NPAD, DH), jnp.float32),
      pltpu.VMEM_SHARED((NPAD, DH), jnp.float32),
      pltpu.SemaphoreType.DMA((RB2,)),
      pltpu.SemaphoreType.DMA((RB2,)),
  ]

  def body(ga, gb, src3, dst3, outa, outb,
           src_v, dst_v, rows_v, accum, table_sp, ssem, gsem):
    c = lax.axis_index("c")
    s = lax.axis_index("s")
    wid = c * NS + s
    r0 = pl.multiple_of(s * RPT, RPT)

    def zero_fill(r, carry):
      for j in range(DH // 16):
        rows_v[0, r, pl.ds(j * 16, 16)] = jnp.zeros((16,), jnp.float32)
      return carry

    def scat_wait(b):
      pltpu.make_async_copy(
          rows_v.at[0], accum.at[dst_v.at[0]], ssem.at[b]).wait()

    def gath_wait(b):
      pltpu.make_async_copy(
          table_sp.at[src_v.at[0]], rows_v.at[0], gsem.at[b]).wait()

    def gath_start(i):
      b = lax.rem(i, RB2)
      pltpu.async_copy(table_sp.at[src_v.at[i]], rows_v.at[b], gsem.at[b])

    pltpu.sync_copy(src3.at[wid], src_v)
    pltpu.sync_copy(dst3.at[wid], dst_v)

    for table, out in ((ga, outa), (gb, outb)):
      lax.fori_loop(0, CH, zero_fill, 0)
      for k in range(RPT // CH):
        pltpu.sync_copy(rows_v.at[0], accum.at[pl.ds(r0 + k * CH, CH)])
      pltpu.sync_copy(table.at[pl.ds(r0, RPT)], table_sp.at[pl.ds(r0, RPT)])
      plsc.subcore_barrier()

      for j in range(GD2):
        gath_start(j)

      def step(i, carry):
        b = lax.rem(i, RB2)
        gath_wait(b)
        pltpu.async_copy(
            rows_v.at[b], accum.at[dst_v.at[i]], ssem.at[b], add=True)
        j = i + GD2

        @pl.when(j < NCHUNK)
        def _():
          @pl.when(j >= RB2)
          def _():
            scat_wait(lax.rem(j, RB2))
          gath_start(j)
        return carry

      lax.fori_loop(0, NCHUNK, step, 0)
      for b in range(RB2):
        scat_wait(b)
      plsc.subcore_barrier()
      pltpu.sync_copy(accum.at[pl.ds(r0, RPT)], out.at[c, pl.ds(r0, RPT)])

  return pl.kernel(
      body,
      out_type=(jax.ShapeDtypeStruct((NC, NPAD, DH), jnp.float32),
                jax.ShapeDtypeStruct((NC, NPAD, DH), jnp.float32)),
      mesh=_mesh(),
      scratch_types=scratch,
      compiler_params=pltpu.CompilerParams(use_tc_tiling_on_sc=False),
  )


NCHUNK_ALL = EPAD // NS // CH


def _sc_deg_agg1():
  rb, gd = 8, 5
  scratch = [
      pltpu.VMEM((NCHUNK_ALL, CH), jnp.int32),
      pltpu.VMEM((NCHUNK, CH), jnp.int32),
      pltpu.VMEM((NCHUNK, CH), jnp.int32),
      pltpu.VMEM((rb, CH, D1), jnp.float32),
      pltpu.VMEM((RPT, D1), jnp.float32),
      pltpu.VMEM((RPT, D1), jnp.float32),
      pltpu.VMEM_SHARED((NPAD, D1), jnp.float32),
      pltpu.VMEM_SHARED((NPAD, D1), jnp.float32),
      pltpu.VMEM_SHARED((NPAD, D1), jnp.float32),
      pltpu.SemaphoreType.DMA((rb,)),
      pltpu.SemaphoreType.DMA((rb,)),
  ]

  def body(x16, src3, dst3, g1_out, s1_out, dsta_v, src_v, dst_v,
           rows_v, work_v, deg_v, degacc, table, s1acc, ssem, gsem):
    c = lax.axis_index("c")
    s = lax.axis_index("s")
    wid = c * NS + s
    r0 = pl.multiple_of(s * RPT, RPT)

    def make_fill(val):
      def fill_row(r, carry):
        rows_v[0, r, :] = jnp.full((16,), val, jnp.float32)
        return carry
      return fill_row

    lax.fori_loop(0, CH, make_fill(0.0), 0)
    for k in range(RPT // CH):
      pltpu.sync_copy(rows_v.at[0], degacc.at[pl.ds(r0 + k * CH, CH)])
      pltpu.sync_copy(rows_v.at[0], s1acc.at[pl.ds(r0 + k * CH, CH)])
    lax.fori_loop(0, CH, make_fill(1.0), 0)
    pltpu.sync_copy(dst3.at[2 * s], dsta_v.at[pl.ds(0, NCHUNK)])
    pltpu.sync_copy(dst3.at[2 * s + 1], dsta_v.at[pl.ds(NCHUNK, NCHUNK)])
    pltpu.sync_copy(src3.at[wid], src_v)
    pltpu.sync_copy(dst3.at[wid], dst_v)
    pltpu.sync_copy(x16.at[pl.ds(r0, RPT)], work_v)
    plsc.subcore_barrier()

    def scat_wait(b):
      pltpu.make_async_copy(
          rows_v.at[0], s1acc.at[pl.ds(0, CH)], ssem.at[b]).wait()

    def deg_step(i, carry):
      b = lax.rem(i, rb)

      @pl.when(i >= rb)
      def _():
        scat_wait(b)
      pltpu.async_copy(
          rows_v.at[0], degacc.at[dsta_v.at[i]], ssem.at[b], add=True)
      return carry

    lax.fori_loop(0, NCHUNK_ALL, deg_step, 0)
    for b in range(rb):
      scat_wait(b)
    plsc.subcore_barrier()

    pltpu.sync_copy(degacc.at[pl.ds(r0, RPT)], deg_v)
    lane = lax.iota(jnp.int32, 16)
    magic = jnp.full((16,), 0x5F3759DF, jnp.int32)

    def norm_row(r, carry):
      d1 = deg_v[r, :] + 1.0
      y = plsc.bitcast(
          magic - lax.shift_right_logical(plsc.bitcast(d1, jnp.int32), 1),
          jnp.float32)
      for _ in range(3):
        y = y * (1.5 - 0.5 * d1 * y * y)
      y = jnp.where(r0 + r < N, y, 0.0)
      work_v[r, :] = jnp.where(lane == 4, y, work_v[r, :] * y)
      return carry

    lax.fori_loop(0, RPT, norm_row, 0)
    pltpu.sync_copy(work_v, table.at[pl.ds(r0, RPT)])

    @pl.when(c == 0)
    def _():
      pltpu.sync_copy(work_v, g1_out.at[pl.ds(r0, RPT)])
    plsc.subcore_barrier()

    def gath_start(i):
      b = lax.rem(i, rb)
      pltpu.async_copy(table.at[src_v.at[i]], rows_v.at[b], gsem.at[b])

    def gath_wait(b):
      pltpu.make_async_copy(
          table.at[pl.ds(0, CH)], rows_v.at[0], gsem.at[b]).wait()

    for j in range(gd):
      gath_start(j)

    def step(i, carry):
      b = lax.rem(i, rb)
      gath_wait(b)
      pltpu.async_copy(
          rows_v.at[b], s1acc.at[dst_v.at[i]], ssem.at[b], add=True)
      j = i + gd

      @pl.when(j < NCHUNK)
      def _():
        @pl.when(j >= rb)
        def _():
          scat_wait(lax.rem(j, rb))
        gath_start(j)
      return carry

    lax.fori_loop(0, NCHUNK, step, 0)
    for b in range(min(rb, NCHUNK)):
      scat_wait(b)
    plsc.subcore_barrier()
    pltpu.sync_copy(s1acc.at[pl.ds(r0, RPT)], s1_out.at[c, pl.ds(r0, RPT)])

  return pl.kernel(
      body,
      out_type=(jax.ShapeDtypeStruct((NPAD, D1), jnp.float32),
                jax.ShapeDtypeStruct((NC, NPAD, D1), jnp.float32)),
      mesh=_mesh(),
      scratch_types=scratch,
      compiler_params=pltpu.CompilerParams(
          use_tc_tiling_on_sc=False, needs_layout_passes=False),
  )


BR = 512
_GRID = NPAD // BR


def _layer1_body(g1_ref, s1_ref, x_ref, w_ref, b_ref, g2a_ref, g2b_ref):
  dinv = g1_ref[:, 4:5]
  a1 = (s1_ref[0] + s1_ref[1]) * dinv + x_ref[...] * (dinv * dinv)
  h1 = jnp.maximum(
      jnp.dot(a1, w_ref[...], preferred_element_type=jnp.float32)
      + b_ref[...], 0.0)
  g2 = h1 * dinv
  g2a_ref[...] = g2[:, :DH]
  g2b_ref[...] = g2[:, DH:]


def _tc_layer1(g1, s1_parts, x16, w1p, b1):
  return pl.pallas_call(
      _layer1_body,
      grid=(_GRID,),
      in_specs=[
          pl.BlockSpec((BR, D1), lambda i: (i, 0)),
          pl.BlockSpec((NC, BR, D1), lambda i: (0, i, 0)),
          pl.BlockSpec((BR, D1), lambda i: (i, 0)),
          pl.BlockSpec((D1, HID), lambda i: (0, 0)),
          pl.BlockSpec((1, HID), lambda i: (0, 0)),
      ],
      out_specs=[
          pl.BlockSpec((BR, DH), lambda i: (i, 0)),
          pl.BlockSpec((BR, DH), lambda i: (i, 0)),
      ],
      out_shape=[
          jax.ShapeDtypeStruct((NPAD, DH), jnp.float32),
          jax.ShapeDtypeStruct((NPAD, DH), jnp.float32),
      ],
  )(g1, s1_parts, x16, w1p, b1)


def _layer2_body(g1_ref, s2a_ref, s2b_ref, g2a_ref, g2b_ref, w_ref, b_ref,
                 y_ref):
  dinv = g1_ref[:, 4:5]
  a2 = jnp.concatenate(
      [s2a_ref[0] + s2a_ref[1] + g2a_ref[...],
       s2b_ref[0] + s2b_ref[1] + g2b_ref[...]], axis=1) * dinv
  y_ref[...] = jnp.maximum(
      jnp.dot(a2, w_ref[...], preferred_element_type=jnp.float32)
      + b_ref[...], 0.0)


def _tc_layer2(g1, s2a_parts, s2b_parts, g2a, g2b, w2, b2):
  return pl.pallas_call(
      _layer2_body,
      grid=(_GRID,),
      in_specs=[
          pl.BlockSpec((BR, D1), lambda i: (i, 0)),
          pl.BlockSpec((NC, BR, DH), lambda i: (0, i, 0)),
          pl.BlockSpec((NC, BR, DH), lambda i: (0, i, 0)),
          pl.BlockSpec((BR, DH), lambda i: (i, 0)),
          pl.BlockSpec((BR, DH), lambda i: (i, 0)),
          pl.BlockSpec((HID, HID), lambda i: (0, 0)),
          pl.BlockSpec((1, HID), lambda i: (0, 0)),
      ],
      out_specs=pl.BlockSpec((BR, HID), lambda i: (i, 0)),
      out_shape=jax.ShapeDtypeStruct((NPAD, HID), jnp.float32),
  )(g1, s2a_parts, s2b_parts, g2a, g2b, w2, b2)


@jax.jit
def _run(x, edge_index, w1, b1, w2, b2):
  pad = jnp.full((EPAD - E,), N, dtype=jnp.int32)
  srcf = jnp.concatenate([edge_index[0].astype(jnp.int32), pad])
  dstf = jnp.concatenate([edge_index[1].astype(jnp.int32), pad])
  srcp = srcf.reshape(NC * NS, NCHUNK, CH)
  dstp = dstf.reshape(NC * NS, NCHUNK, CH)
  x16 = jnp.zeros((NPAD, D1), jnp.float32).at[:N, :4].set(x)
  w1p = jnp.zeros((D1, HID), jnp.float32).at[:4].set(w1)

  g1, s1_parts = _sc_deg_agg1()(x16, srcp, dstp)
  g2a, g2b = _tc_layer1(g1, s1_parts, x16, w1p, b1.reshape(1, HID))
  s2a_parts, s2b_parts = _sc_agg2()(g2a, g2b, srcp, dstp)
  y = _tc_layer2(g1, s2a_parts, s2b_parts, g2a, g2b, w2,
                 b2.reshape(1, HID))
  return y[:N]


def kernel(x, edge_index, W1, b1, W2, b2):
  return _run(x, edge_index, W1, b1, W2, b2)


# speedup vs baseline: 1.0105x; 1.0048x over previous
import functools

import jax
import jax.numpy as jnp
from jax import lax
from jax.experimental import pallas as pl
from jax.experimental.pallas import tpu as pltpu
from jax.experimental.pallas import tpu_sc as plsc

N = 10000
E = 320000
HID = 128
D1 = 16
DH = 64

NC, NS = 2, 16
NPAD = 10240
EPAD = 327680
EPT = EPAD // (NC * NS)
CH = 128
NCHUNK = EPT // CH
RPT = NPAD // NS

_mesh = functools.partial(
    plsc.VectorSubcoreMesh, core_axis_name="c", subcore_axis_name="s",
    num_cores=NC, num_subcores=NS)


RB2, GD2 = 3, 2


def _sc_agg2():
  scratch = [
      pltpu.VMEM((NCHUNK, CH), jnp.int32),
      pltpu.VMEM((NCHUNK, CH), jnp.int32),
      pltpu.VMEM((RB2, CH, DH), jnp.float32),
      pltpu.VMEM_SHARED((NPAD, DH), jnp.float32),
      pltpu.VMEM_SHARED((NPAD, DH), jnp.float32),
      pltpu.SemaphoreType.DMA((RB2,)),
      pltpu.SemaphoreType.DMA((RB2,)),
  ]

  def body(ga, gb, src3, dst3, outa, outb,
           src_v, dst_v, rows_v, accum, table_sp, ssem, gsem):
    c = lax.axis_index("c")
    s = lax.axis_index("s")
    wid = c * NS + s
    r0 = pl.multiple_of(s * RPT, RPT)

    def zero_fill(r, carry):
      for j in range(DH // 16):
        rows_v[0, r, pl.ds(j * 16, 16)] = jnp.zeros((16,), jnp.float32)
      return carry

    def scat_wait(b):
      pltpu.make_async_copy(
          rows_v.at[0], accum.at[dst_v.at[0]], ssem.at[b]).wait()

    def gath_wait(b):
      pltpu.make_async_copy(
          table_sp.at[src_v.at[0]], rows_v.at[0], gsem.at[b]).wait()

    def gath_start(i):
      b = lax.rem(i, RB2)
      pltpu.async_copy(table_sp.at[src_v.at[i]], rows_v.at[b], gsem.at[b])

    pltpu.sync_copy(src3.at[wid], src_v)
    pltpu.sync_copy(dst3.at[wid], dst_v)

    for table, out in ((ga, outa), (gb, outb)):
      lax.fori_loop(0, CH, zero_fill, 0)
      for k in range(RPT // CH):
        pltpu.sync_copy(rows_v.at[0], accum.at[pl.ds(r0 + k * CH, CH)])
      pltpu.sync_copy(table.at[pl.ds(r0, RPT)], table_sp.at[pl.ds(r0, RPT)])
      plsc.subcore_barrier()

      for j in range(GD2):
        gath_start(j)

      def step(i, carry):
        b = lax.rem(i, RB2)
        gath_wait(b)
        pltpu.async_copy(
            rows_v.at[b], accum.at[dst_v.at[i]], ssem.at[b], add=True)
        j = i + GD2

        @pl.when(j < NCHUNK)
        def _():
          @pl.when(j >= RB2)
          def _():
            scat_wait(lax.rem(j, RB2))
          gath_start(j)
        return carry

      lax.fori_loop(0, NCHUNK, step, 0)
      for b in range(RB2):
        scat_wait(b)
      plsc.subcore_barrier()
      pltpu.sync_copy(accum.at[pl.ds(r0, RPT)], out.at[c, pl.ds(r0, RPT)])

  return pl.kernel(
      body,
      out_type=(jax.ShapeDtypeStruct((NC, NPAD, DH), jnp.float32),
                jax.ShapeDtypeStruct((NC, NPAD, DH), jnp.float32)),
      mesh=_mesh(),
      scratch_types=scratch,
      compiler_params=pltpu.CompilerParams(use_tc_tiling_on_sc=False),
  )


NCHUNK_ALL = EPAD // NS // CH


def _sc_deg_agg1():
  rb, gd = 8, 5
  scratch = [
      pltpu.VMEM((NCHUNK_ALL, CH), jnp.int32),
      pltpu.VMEM((NCHUNK, CH), jnp.int32),
      pltpu.VMEM((NCHUNK, CH), jnp.int32),
      pltpu.VMEM((rb, CH, D1), jnp.float32),
      pltpu.VMEM((RPT, D1), jnp.float32),
      pltpu.VMEM((RPT, D1), jnp.float32),
      pltpu.VMEM_SHARED((NPAD, D1), jnp.float32),
      pltpu.VMEM_SHARED((NPAD, D1), jnp.float32),
      pltpu.VMEM_SHARED((NPAD, D1), jnp.float32),
      pltpu.SemaphoreType.DMA((rb,)),
      pltpu.SemaphoreType.DMA((rb,)),
  ]

  def body(x16, src3, dst3, g1_out, s1_out, dsta_v, src_v, dst_v,
           rows_v, work_v, deg_v, degacc, table, s1acc, ssem, gsem):
    c = lax.axis_index("c")
    s = lax.axis_index("s")
    wid = c * NS + s
    r0 = pl.multiple_of(s * RPT, RPT)

    def make_fill(val):
      def fill_row(r, carry):
        rows_v[0, r, :] = jnp.full((16,), val, jnp.float32)
        return carry
      return fill_row

    lax.fori_loop(0, CH, make_fill(0.0), 0)
    for k in range(RPT // CH):
      pltpu.sync_copy(rows_v.at[0], degacc.at[pl.ds(r0 + k * CH, CH)])
      pltpu.sync_copy(rows_v.at[0], s1acc.at[pl.ds(r0 + k * CH, CH)])
    lax.fori_loop(0, CH, make_fill(1.0), 0)
    pltpu.sync_copy(dst3.at[2 * s], dsta_v.at[pl.ds(0, NCHUNK)])
    pltpu.sync_copy(dst3.at[2 * s + 1], dsta_v.at[pl.ds(NCHUNK, NCHUNK)])
    pltpu.sync_copy(src3.at[wid], src_v)
    pltpu.sync_copy(dst3.at[wid], dst_v)
    pltpu.sync_copy(x16.at[pl.ds(r0, RPT)], work_v)
    plsc.subcore_barrier()

    def scat_wait(b):
      pltpu.make_async_copy(
          rows_v.at[0], s1acc.at[pl.ds(0, CH)], ssem.at[b]).wait()

    def deg_step(i, carry):
      b = lax.rem(i, rb)

      @pl.when(i >= rb)
      def _():
        scat_wait(b)
      pltpu.async_copy(
          rows_v.at[0], degacc.at[dsta_v.at[i]], ssem.at[b], add=True)
      return carry

    lax.fori_loop(0, NCHUNK_ALL, deg_step, 0)
    for b in range(rb):
      scat_wait(b)
    plsc.subcore_barrier()

    pltpu.sync_copy(degacc.at[pl.ds(r0, RPT)], deg_v)
    lane = lax.iota(jnp.int32, 16)
    magic = jnp.full((16,), 0x5F3759DF, jnp.int32)

    def norm_row(r, carry):
      d1 = deg_v[r, :] + 1.0
      y = plsc.bitcast(
          magic - lax.shift_right_logical(plsc.bitcast(d1, jnp.int32), 1),
          jnp.float32)
      for _ in range(3):
        y = y * (1.5 - 0.5 * d1 * y * y)
      y = jnp.where(r0 + r < N, y, 0.0)
      work_v[r, :] = jnp.where(lane == 4, y, work_v[r, :] * y)
      return carry

    lax.fori_loop(0, RPT, norm_row, 0)
    pltpu.sync_copy(work_v, table.at[pl.ds(r0, RPT)])

    @pl.when(c == 0)
    def _():
      pltpu.sync_copy(work_v, g1_out.at[pl.ds(r0, RPT)])
    plsc.subcore_barrier()

    def gath_start(i):
      b = lax.rem(i, rb)
      pltpu.async_copy(table.at[src_v.at[i]], rows_v.at[b], gsem.at[b])

    def gath_wait(b):
      pltpu.make_async_copy(
          table.at[pl.ds(0, CH)], rows_v.at[0], gsem.at[b]).wait()

    for j in range(gd):
      gath_start(j)

    def step(i, carry):
      b = lax.rem(i, rb)
      gath_wait(b)
      pltpu.async_copy(
          rows_v.at[b], s1acc.at[dst_v.at[i]], ssem.at[b], add=True)
      j = i + gd

      @pl.when(j < NCHUNK)
      def _():
        @pl.when(j >= rb)
        def _():
          scat_wait(lax.rem(j, rb))
        gath_start(j)
      return carry

    lax.fori_loop(0, NCHUNK, step, 0)
    for b in range(min(rb, NCHUNK)):
      scat_wait(b)
    plsc.subcore_barrier()
    pltpu.sync_copy(s1acc.at[pl.ds(r0, RPT)], s1_out.at[c, pl.ds(r0, RPT)])

  return pl.kernel(
      body,
      out_type=(jax.ShapeDtypeStruct((NPAD, D1), jnp.float32),
                jax.ShapeDtypeStruct((NC, NPAD, D1), jnp.float32)),
      mesh=_mesh(),
      scratch_types=scratch,
      compiler_params=pltpu.CompilerParams(
          use_tc_tiling_on_sc=False, needs_layout_passes=False),
  )


BR = 512
_GRID = NPAD // BR


def _layer1_body(g1_ref, s1_ref, x_ref, w_ref, b_ref, g2a_ref, g2b_ref):
  dinv = g1_ref[:, 4:5]
  a1 = (s1_ref[0] + s1_ref[1]) * dinv + x_ref[...] * (dinv * dinv)
  h1 = jnp.maximum(
      jnp.dot(a1, w_ref[...], preferred_element_type=jnp.float32)
      + b_ref[...], 0.0)
  g2 = h1 * dinv
  g2a_ref[...] = g2[:, :DH]
  g2b_ref[...] = g2[:, DH:]


def _tc_layer1(g1, s1_parts, x16, w1p, b1):
  return pl.pallas_call(
      _layer1_body,
      grid=(_GRID,),
      in_specs=[
          pl.BlockSpec((BR, D1), lambda i: (i, 0)),
          pl.BlockSpec((NC, BR, D1), lambda i: (0, i, 0)),
          pl.BlockSpec((BR, D1), lambda i: (i, 0)),
          pl.BlockSpec((D1, HID), lambda i: (0, 0)),
          pl.BlockSpec((1, HID), lambda i: (0, 0)),
      ],
      out_specs=[
          pl.BlockSpec((BR, DH), lambda i: (i, 0)),
          pl.BlockSpec((BR, DH), lambda i: (i, 0)),
      ],
      out_shape=[
          jax.ShapeDtypeStruct((NPAD, DH), jnp.float32),
          jax.ShapeDtypeStruct((NPAD, DH), jnp.float32),
      ],
  )(g1, s1_parts, x16, w1p, b1)


def _layer2_body(g1_ref, s2a_ref, s2b_ref, g2a_ref, g2b_ref, w_ref, b_ref,
                 y_ref):
  dinv = g1_ref[:, 4:5]
  a2 = jnp.concatenate(
      [s2a_ref[0] + s2a_ref[1] + g2a_ref[...],
       s2b_ref[0] + s2b_ref[1] + g2b_ref[...]], axis=1) * dinv
  y_ref[...] = jnp.maximum(
      jnp.dot(a2, w_ref[...], preferred_element_type=jnp.float32)
      + b_ref[...], 0.0)


BR2 = 400


def _tc_layer2(g1, s2a_parts, s2b_parts, g2a, g2b, w2, b2):
  return pl.pallas_call(
      _layer2_body,
      grid=(N // BR2,),
      in_specs=[
          pl.BlockSpec((BR2, D1), lambda i: (i, 0)),
          pl.BlockSpec((NC, BR2, DH), lambda i: (0, i, 0)),
          pl.BlockSpec((NC, BR2, DH), lambda i: (0, i, 0)),
          pl.BlockSpec((BR2, DH), lambda i: (i, 0)),
          pl.BlockSpec((BR2, DH), lambda i: (i, 0)),
          pl.BlockSpec((HID, HID), lambda i: (0, 0)),
          pl.BlockSpec((1, HID), lambda i: (0, 0)),
      ],
      out_specs=pl.BlockSpec((BR2, HID), lambda i: (i, 0)),
      out_shape=jax.ShapeDtypeStruct((N, HID), jnp.float32),
  )(g1, s2a_parts, s2b_parts, g2a, g2b, w2, b2)


@jax.jit
def _run(x, edge_index, w1, b1, w2, b2):
  pad = jnp.full((EPAD - E,), N, dtype=jnp.int32)
  srcf = jnp.concatenate([edge_index[0].astype(jnp.int32), pad])
  dstf = jnp.concatenate([edge_index[1].astype(jnp.int32), pad])
  srcp = srcf.reshape(NC * NS, NCHUNK, CH)
  dstp = dstf.reshape(NC * NS, NCHUNK, CH)
  x16 = jnp.zeros((NPAD, D1), jnp.float32).at[:N, :4].set(x)
  w1p = jnp.zeros((D1, HID), jnp.float32).at[:4].set(w1)

  g1, s1_parts = _sc_deg_agg1()(x16, srcp, dstp)
  g2a, g2b = _tc_layer1(g1, s1_parts, x16, w1p, b1.reshape(1, HID))
  s2a_parts, s2b_parts = _sc_agg2()(g2a, g2b, srcp, dstp)
  return _tc_layer2(g1, s2a_parts, s2b_parts, g2a, g2b, w2,
                    b2.reshape(1, HID))


def kernel(x, edge_index, W1, b1, W2, b2):
  return _run(x, edge_index, W1, b1, W2, b2)
